# Initial kernel scaffold; baseline (speedup 1.0000x reference)
import jax
import jax.numpy as jnp
from jax.experimental import pallas as pl


def kernel(x, edge_index, W1, a_src1, a_dst1, b1, W2, a_src2, a_dst2, b2):
    raise NotImplementedError("write your pallas kernel here")


# speedup vs baseline: 7.6723x; 7.6723x over previous
import functools

import jax
import jax.numpy as jnp
from jax import lax
from jax.experimental import pallas as pl
from jax.experimental.pallas import tpu as pltpu
from jax.experimental.pallas import tpu_sc as plsc

N = 10000
E = 320000
IN = 128
HID = 256
OUT = 128

NC = 2
NS = 16
L = 16

NP = 10240
EP = E // NS
NPS = NP // NS
CH = 80
NCH = EP // CH
CW = 32


def _tc1_body(x_ref, w_ref, asrc_ref, adst_ref, hh_ref, s2_ref):
    h = jnp.dot(x_ref[...], w_ref[...], preferred_element_type=jnp.float32)
    for q in range(HID // CW):
        hh_ref[q] = h[:, q * CW:(q + 1) * CW]
    s2_ref[...] = jnp.stack([h @ asrc_ref[...], h @ adst_ref[...]])


def _tc2_body(p_ref, b1_ref, w_ref, asrc_ref, adst_ref, hh_ref, s2_ref):
    p = jnp.concatenate([p_ref[q] for q in range(HID // CW)], axis=1)
    z = jnp.maximum(p + b1_ref[...][None, :], 0.0)
    h = jnp.dot(z, w_ref[...], preferred_element_type=jnp.float32)
    for q in range(OUT // CW):
        hh_ref[q] = h[:, q * CW:(q + 1) * CW]
    s2_ref[...] = jnp.stack([h @ asrc_ref[...], h @ adst_ref[...]])


_TCG = 4
_BN = NP // _TCG

_tc1 = pl.pallas_call(
    _tc1_body,
    grid=(_TCG,),
    in_specs=[
        pl.BlockSpec((_BN, IN), lambda i: (i, 0)),
        pl.BlockSpec((IN, HID), lambda i: (0, 0)),
        pl.BlockSpec((HID,), lambda i: (0,)),
        pl.BlockSpec((HID,), lambda i: (0,)),
    ],
    out_specs=(
        pl.BlockSpec((HID // CW, _BN, CW), lambda i: (0, i, 0)),
        pl.BlockSpec((2, _BN), lambda i: (0, i)),
    ),
    out_shape=(
        jax.ShapeDtypeStruct((HID // CW, NP, CW), jnp.float32),
        jax.ShapeDtypeStruct((2, NP), jnp.float32),
    ),
)

_tc2 = pl.pallas_call(
    _tc2_body,
    grid=(_TCG,),
    in_specs=[
        pl.BlockSpec((HID // CW, _BN, CW), lambda i: (0, i, 0)),
        pl.BlockSpec((HID,), lambda i: (0,)),
        pl.BlockSpec((HID, OUT), lambda i: (0, 0)),
        pl.BlockSpec((OUT,), lambda i: (0,)),
        pl.BlockSpec((OUT,), lambda i: (0,)),
    ],
    out_specs=(
        pl.BlockSpec((OUT // CW, _BN, CW), lambda i: (0, i, 0)),
        pl.BlockSpec((2, _BN), lambda i: (0, i)),
    ),
    out_shape=(
        jax.ShapeDtypeStruct((OUT // CW, NP, CW), jnp.float32),
        jax.ShapeDtypeStruct((2, NP), jnp.float32),
    ),
)


@functools.cache
def _make_sc_gat(P):
    mesh = plsc.VectorSubcoreMesh(
        core_axis_name="c", subcore_axis_name="s",
        num_cores=NC, num_subcores=NS)

    @functools.partial(
        pl.kernel,
        out_type=(
            jax.ShapeDtypeStruct((2 * P * NP, CW), jnp.float32),
            jax.ShapeDtypeStruct((NC, NS + 1, NP), jnp.float32),
        ),
        mesh=mesh,
        compiler_params=pltpu.CompilerParams(
            needs_layout_passes=False, use_tc_tiling_on_sc=False),
        scratch_types=[
            pltpu.VMEM((EP,), jnp.int32),
            pltpu.VMEM((EP,), jnp.int32),
            pltpu.VMEM((EP,), jnp.float32),
            pltpu.VMEM((NP,), jnp.float32),
            pltpu.VMEM((NP,), jnp.float32),
            pltpu.VMEM((NP,), jnp.float32),
            pltpu.VMEM((CH, CW), jnp.float32),
            pltpu.VMEM((CH,), jnp.int32),
            pltpu.VMEM((CH,), jnp.int32),
            pltpu.VMEM_SHARED((NP, CW), jnp.float32),
            pltpu.SemaphoreType.DMA,
        ],
    )
    def sc_gat(src_hbm, dst_hbm, s2_hbm, hh_hbm, out_hbm, red_hbm,
               sv, dv, ebuf, ssrc, sdst, dbuf, rows, svc, dvc,
               acc, sem):
        c = lax.axis_index("c")
        s = lax.axis_index("s")
        base = s * EP
        myoff = s * NPS
        zero16 = jnp.zeros((L,), jnp.float32)

        pltpu.sync_copy(src_hbm.at[pl.ds(base, EP)], sv)
        pltpu.sync_copy(dst_hbm.at[pl.ds(base, EP)], dv)
        pltpu.sync_copy(s2_hbm.at[0], ssrc)
        pltpu.sync_copy(s2_hbm.at[1], sdst)

        def zero_dbuf(i, _):
            dbuf[pl.ds(i * L, L)] = zero16
            return 0
        lax.fori_loop(0, NP // L, zero_dbuf, 0)

        def p1(i, _):
            sl = pl.ds(i * L, L)
            a = plsc.load_gather(ssrc, [sv[sl]])
            b = plsc.load_gather(sdst, [dv[sl]])
            t = a + b
            ex = jnp.exp(jnp.where(t >= 0.0, t, t * 0.2))
            ebuf[sl] = ex
            plsc.addupdate_scatter(dbuf, [dv[sl]], ex)
            return 0
        lax.fori_loop(0, EP // L, p1, 0)

        pltpu.sync_copy(dbuf, red_hbm.at[c, s])

        def zrow(j, _):
            def zcol(k, _):
                rows[j, pl.ds(k * L, L)] = zero16
                return 0
            return lax.fori_loop(0, CW // L, zcol, 0)
        lax.fori_loop(0, CH, zrow, 0)

        plsc.subcore_barrier()

        def zs(k, _):
            ssrc[pl.ds(k * L, L)] = zero16
            return 0
        lax.fori_loop(0, NPS // L, zs, 0)

        def rt(t, _):
            pltpu.sync_copy(red_hbm.at[c, t, pl.ds(myoff, NPS)],
                            sdst.at[pl.ds(0, NPS)])
            def radd(k, _):
                sl = pl.ds(k * L, L)
                ssrc[sl] = ssrc[sl] + sdst[sl]
                return 0
            return lax.fori_loop(0, NPS // L, radd, 0)
        lax.fori_loop(0, NS, rt, 0)

        pltpu.sync_copy(ssrc.at[pl.ds(0, NPS)],
                        red_hbm.at[c, NS, pl.ds(myoff, NPS)])
        plsc.subcore_barrier()
        pltpu.sync_copy(red_hbm.at[c, NS], dbuf)

        def nrm(i, _):
            sl = pl.ds(i * L, L)
            den = plsc.load_gather(dbuf, [dv[sl]])
            ebuf[sl] = ebuf[sl] / (den + 1e-16)
            return 0
        lax.fori_loop(0, EP // L, nrm, 0)

        for p in range(P):
            coff = (c * P + p) * NP

            def zacc(i, _):
                pltpu.sync_copy(rows, acc.at[pl.ds(myoff + i * CH, CH)])
                return 0
            lax.fori_loop(0, NPS // CH, zacc, 0)
            plsc.subcore_barrier()

            def p2(i, _):
                ch0 = i * CH

                def mk(k, _):
                    sl = pl.ds(ch0 + k * L, L)
                    slc = pl.ds(k * L, L)
                    svc[slc] = sv[sl] + coff
                    dvc[slc] = dv[sl]
                    return 0
                lax.fori_loop(0, CH // L, mk, 0)

                pltpu.async_copy(hh_hbm.at[svc], rows, sem).wait()

                def se(g, _):
                    av = ebuf[pl.ds(ch0 + g * L, L)]
                    for j in range(L):
                        aj = av[j]
                        row = g * L + j

                        def scl(k, _, row=row, aj=aj):
                            sl = pl.ds(k * L, L)
                            rows[row, sl] = rows[row, sl] * aj
                            return 0
                        lax.fori_loop(0, CW // L, scl, 0)
                    return 0
                lax.fori_loop(0, CH // L, se, 0)

                pltpu.sync_copy(rows, acc.at[dvc], add=True)
                return 0
            lax.fori_loop(0, NCH, p2, 0)

            plsc.subcore_barrier()
            pltpu.sync_copy(acc.at[pl.ds(myoff, NPS)],
                            out_hbm.at[pl.ds(coff + myoff, NPS)])
            if p + 1 < P:
                def zrow2(j, _):
                    def zcol2(k, _):
                        rows[j, pl.ds(k * L, L)] = zero16
                        return 0
                    return lax.fori_loop(0, CW // L, zcol2, 0)
                lax.fori_loop(0, CH, zrow2, 0)
                plsc.subcore_barrier()

    return sc_gat


def kernel(x, edge_index, W1, a_src1, a_dst1, b1, W2, a_src2, a_dst2, b2):
    xp = jnp.pad(x, ((0, NP - N), (0, 0)))
    src = edge_index[0]
    dst = edge_index[1]

    hh1, s21 = _tc1(xp, W1, a_src1, a_dst1)
    nb1 = HID // CW
    o1, _ = _make_sc_gat(nb1 // 2)(src, dst, s21, hh1.reshape(nb1 * NP, CW))

    hh2, s22 = _tc2(o1.reshape(nb1, NP, CW), b1, W2, a_src2, a_dst2)
    nb2 = OUT // CW
    o2, _ = _make_sc_gat(nb2 // 2)(src, dst, s22, hh2.reshape(nb2 * NP, CW))

    o2 = o2.reshape(nb2, NP, CW)
    out = jnp.concatenate([o2[q] for q in range(nb2)], axis=1)[:N] + b2
    return out


# speedup vs baseline: 17.6727x; 2.3034x over previous
import functools

import jax
import jax.numpy as jnp
from jax import lax
from jax.experimental import pallas as pl
from jax.experimental.pallas import tpu as pltpu
from jax.experimental.pallas import tpu_sc as plsc

N = 10000
E = 320000
IN = 128
HID = 256
OUT = 128

NC = 2
NS = 16
L = 16

NP = 10240
EP = E // NS
NPS = NP // NS
CH = 80
NCH = EP // CH
NB = 5
CW = 32


def _tc1_body(x_ref, w_ref, asrc_ref, adst_ref, hh_ref, s2_ref):
    h = jnp.dot(x_ref[...], w_ref[...], preferred_element_type=jnp.float32)
    for q in range(HID // CW):
        hh_ref[q] = h[:, q * CW:(q + 1) * CW]
    s2_ref[...] = jnp.stack([h @ asrc_ref[...], h @ adst_ref[...]])


def _tc2_body(p_ref, b1_ref, w_ref, asrc_ref, adst_ref, hh_ref, s2_ref):
    p = jnp.concatenate([p_ref[q] for q in range(HID // CW)], axis=1)
    z = jnp.maximum(p + b1_ref[...][None, :], 0.0)
    h = jnp.dot(z, w_ref[...], preferred_element_type=jnp.float32)
    for q in range(OUT // CW):
        hh_ref[q] = h[:, q * CW:(q + 1) * CW]
    s2_ref[...] = jnp.stack([h @ asrc_ref[...], h @ adst_ref[...]])


_TCG = 4
_BN = NP // _TCG

_tc1 = pl.pallas_call(
    _tc1_body,
    grid=(_TCG,),
    in_specs=[
        pl.BlockSpec((_BN, IN), lambda i: (i, 0)),
        pl.BlockSpec((IN, HID), lambda i: (0, 0)),
        pl.BlockSpec((HID,), lambda i: (0,)),
        pl.BlockSpec((HID,), lambda i: (0,)),
    ],
    out_specs=(
        pl.BlockSpec((HID // CW, _BN, CW), lambda i: (0, i, 0)),
        pl.BlockSpec((2, _BN), lambda i: (0, i)),
    ),
    out_shape=(
        jax.ShapeDtypeStruct((HID // CW, NP, CW), jnp.float32),
        jax.ShapeDtypeStruct((2, NP), jnp.float32),
    ),
)

_tc2 = pl.pallas_call(
    _tc2_body,
    grid=(_TCG,),
    in_specs=[
        pl.BlockSpec((HID // CW, _BN, CW), lambda i: (0, i, 0)),
        pl.BlockSpec((HID,), lambda i: (0,)),
        pl.BlockSpec((HID, OUT), lambda i: (0, 0)),
        pl.BlockSpec((OUT,), lambda i: (0,)),
        pl.BlockSpec((OUT,), lambda i: (0,)),
    ],
    out_specs=(
        pl.BlockSpec((OUT // CW, _BN, CW), lambda i: (0, i, 0)),
        pl.BlockSpec((2, _BN), lambda i: (0, i)),
    ),
    out_shape=(
        jax.ShapeDtypeStruct((OUT // CW, NP, CW), jnp.float32),
        jax.ShapeDtypeStruct((2, NP), jnp.float32),
    ),
)


@functools.cache
def _make_sc_gat(P):
    mesh = plsc.VectorSubcoreMesh(
        core_axis_name="c", subcore_axis_name="s",
        num_cores=NC, num_subcores=NS)

    @functools.partial(
        pl.kernel,
        out_type=(
            jax.ShapeDtypeStruct((2 * P * NP, CW), jnp.float32),
            jax.ShapeDtypeStruct((NC, NS + 1, NP), jnp.float32),
        ),
        mesh=mesh,
        compiler_params=pltpu.CompilerParams(
            needs_layout_passes=False, use_tc_tiling_on_sc=False),
        scratch_types=[
            pltpu.VMEM((EP,), jnp.int32),
            pltpu.VMEM((EP,), jnp.int32),
            pltpu.VMEM((EP,), jnp.float32),
            pltpu.VMEM((NP,), jnp.float32),
            pltpu.VMEM((NP,), jnp.float32),
            pltpu.VMEM((NP,), jnp.float32),
            *[pltpu.VMEM((CH, CW), jnp.float32) for _ in range(NB)],
            *[pltpu.VMEM((CH,), jnp.int32) for _ in range(NB)],
            *[pltpu.VMEM((CH,), jnp.int32) for _ in range(NB)],
            pltpu.VMEM_SHARED((NP, CW), jnp.float32),
            *[pltpu.SemaphoreType.DMA for _ in range(2 * NB)],
        ],
    )
    def sc_gat(src_hbm, dst_hbm, s2_hbm, hh_hbm, out_hbm, red_hbm,
               sv, dv, ebuf, ssrc, sdst, dbuf, *rest):
        rows_b = rest[0:NB]
        svc_b = rest[NB:2 * NB]
        dvc_b = rest[2 * NB:3 * NB]
        acc = rest[3 * NB]
        sem_g = rest[3 * NB + 1:3 * NB + 1 + NB]
        sem_s = rest[3 * NB + 1 + NB:3 * NB + 1 + 2 * NB]
        rows = rows_b[0]
        c = lax.axis_index("c")
        s = lax.axis_index("s")
        base = s * EP
        myoff = s * NPS
        zero16 = jnp.zeros((L,), jnp.float32)

        pltpu.sync_copy(src_hbm.at[pl.ds(base, EP)], sv)
        pltpu.sync_copy(dst_hbm.at[pl.ds(base, EP)], dv)
        pltpu.sync_copy(s2_hbm.at[0], ssrc)
        pltpu.sync_copy(s2_hbm.at[1], sdst)

        def zero_dbuf(i, _):
            dbuf[pl.ds(i * L, L)] = zero16
            return 0
        lax.fori_loop(0, NP // L, zero_dbuf, 0)

        def p1(i, _):
            sl = pl.ds(i * L, L)
            a = plsc.load_gather(ssrc, [sv[sl]])
            b = plsc.load_gather(sdst, [dv[sl]])
            t = a + b
            ex = jnp.exp(jnp.where(t >= 0.0, t, t * 0.2))
            ebuf[sl] = ex
            plsc.addupdate_scatter(dbuf, [dv[sl]], ex)
            return 0
        lax.fori_loop(0, EP // L, p1, 0)

        pltpu.sync_copy(dbuf, red_hbm.at[c, s])

        def zrow(j, _):
            def zcol(k, _):
                rows[j, pl.ds(k * L, L)] = zero16
                return 0
            return lax.fori_loop(0, CW // L, zcol, 0)
        lax.fori_loop(0, CH, zrow, 0)

        plsc.subcore_barrier()

        def zs(k, _):
            ssrc[pl.ds(k * L, L)] = zero16
            return 0
        lax.fori_loop(0, NPS // L, zs, 0)

        def rt(t, _):
            pltpu.sync_copy(red_hbm.at[c, t, pl.ds(myoff, NPS)],
                            sdst.at[pl.ds(0, NPS)])
            def radd(k, _):
                sl = pl.ds(k * L, L)
                ssrc[sl] = ssrc[sl] + sdst[sl]
                return 0
            return lax.fori_loop(0, NPS // L, radd, 0)
        lax.fori_loop(0, NS, rt, 0)

        pltpu.sync_copy(ssrc.at[pl.ds(0, NPS)],
                        red_hbm.at[c, NS, pl.ds(myoff, NPS)])
        plsc.subcore_barrier()
        pltpu.sync_copy(red_hbm.at[c, NS], dbuf)

        def nrm(i, _):
            sl = pl.ds(i * L, L)
            den = plsc.load_gather(dbuf, [dv[sl]])
            ebuf[sl] = ebuf[sl] / (den + 1e-16)
            return 0
        lax.fori_loop(0, EP // L, nrm, 0)

        for p in range(P):
            coff = (c * P + p) * NP

            def build(j, svcb, dvcb):
                ch0 = j * CH
                for k in range(CH // L):
                    sl = pl.ds(ch0 + k * L, L)
                    slc = pl.ds(k * L, L)
                    svcb[slc] = sv[sl] + coff
                    dvcb[slc] = dv[sl]

            def scale(j, rref):
                def se(g, _):
                    av = ebuf[pl.ds(j * CH + g * L, L)]
                    for jl in range(L):
                        aj = av[jl]
                        row = g * L + jl
                        for k in range(CW // L):
                            sl = pl.ds(k * L, L)
                            rref[row, sl] = rref[row, sl] * aj
                    return 0
                lax.fori_loop(0, CH // L, se, 0)

            def zrow2(j, _):
                for k in range(CW // L):
                    rows[j, pl.ds(k * L, L)] = zero16
                return 0
            lax.fori_loop(0, CH, zrow2, 0)

            def zacc(i, _):
                pltpu.sync_copy(rows, acc.at[pl.ds(myoff + i * CH, CH)])
                return 0
            lax.fori_loop(0, NPS // CH, zacc, 0)
            plsc.subcore_barrier()

            build(0, svc_b[0], dvc_b[0])
            pltpu.async_copy(hh_hbm.at[svc_b[0]], rows_b[0], sem_g[0])
            build(1, svc_b[1], dvc_b[1])
            pltpu.async_copy(hh_hbm.at[svc_b[1]], rows_b[1], sem_g[1])

            def p2(jj, _):
                for b in range(NB):
                    j = jj * NB + b
                    gb = (b + 2) % NB

                    @pl.when(jnp.logical_and(j >= NB - 2, j + 2 < NCH))
                    def _():
                        pltpu.make_async_copy(
                            rows_b[gb], acc.at[dvc_b[gb]], sem_s[gb]).wait()

                    @pl.when(j + 2 < NCH)
                    def _():
                        build(j + 2, svc_b[gb], dvc_b[gb])
                        pltpu.async_copy(
                            hh_hbm.at[svc_b[gb]], rows_b[gb], sem_g[gb])

                    pltpu.make_async_copy(
                        hh_hbm.at[svc_b[b]], rows_b[b], sem_g[b]).wait()
                    scale(j, rows_b[b])
                    pltpu.async_copy(
                        rows_b[b], acc.at[dvc_b[b]], sem_s[b], add=True)
                return 0
            lax.fori_loop(0, NCH // NB, p2, 0)

            for b in range(NB):
                pltpu.make_async_copy(
                    rows_b[b], acc.at[dvc_b[b]], sem_s[b]).wait()

            plsc.subcore_barrier()
            pltpu.sync_copy(acc.at[pl.ds(myoff, NPS)],
                            out_hbm.at[pl.ds(coff + myoff, NPS)])
            if p + 1 < P:
                plsc.subcore_barrier()

    return sc_gat


def kernel(x, edge_index, W1, a_src1, a_dst1, b1, W2, a_src2, a_dst2, b2):
    xp = jnp.pad(x, ((0, NP - N), (0, 0)))
    src = edge_index[0]
    dst = edge_index[1]

    hh1, s21 = _tc1(xp, W1, a_src1, a_dst1)
    nb1 = HID // CW
    o1, _ = _make_sc_gat(nb1 // 2)(src, dst, s21, hh1.reshape(nb1 * NP, CW))

    hh2, s22 = _tc2(o1.reshape(nb1, NP, CW), b1, W2, a_src2, a_dst2)
    nb2 = OUT // CW
    o2, _ = _make_sc_gat(nb2 // 2)(src, dst, s22, hh2.reshape(nb2 * NP, CW))

    o2 = o2.reshape(nb2, NP, CW)
    out = jnp.concatenate([o2[q] for q in range(nb2)], axis=1)[:N] + b2
    return out
